# initial kernel scaffold (unmeasured)
import jax
import jax.numpy as jnp
from jax import lax
from jax.experimental import pallas as pl
from jax.experimental.pallas import tpu as pltpu

N_DEV = 4
M_CHUNK = 1024


def kernel(x, w_mat, scale_x, scale_w):
    m_tot, k_per = x.shape
    _, n = w_mat.shape
    m_per = m_tot // N_DEV

    def body(x_ref, w_ref, sx_ref, sw_ref, out_ref,
             rbuf, sbuf, send_sems, recv_sems):
        me = lax.axis_index("i")
        right = lax.rem(me + 1, N_DEV)
        left = lax.rem(me + N_DEV - 1, N_DEV)

        barrier_sem = pltpu.get_barrier_semaphore()
        for nbr in (left, right):
            pl.semaphore_signal(
                barrier_sem, inc=1,
                device_id=(nbr,), device_id_type=pl.DeviceIdType.MESH,
            )
        pl.semaphore_wait(barrier_sem, 2)

        w_bf = w_ref[...].astype(jnp.bfloat16)

        def pchunk(c):
            xa = x_ref[pl.ds(c * m_per, m_per), :].astype(jnp.bfloat16)
            return jnp.dot(xa, w_bf, preferred_element_type=jnp.float32)

        sbuf[...] = pchunk(lax.rem(me + N_DEV - 1, N_DEV))
        rdma0 = pltpu.make_async_remote_copy(
            src_ref=sbuf, dst_ref=rbuf.at[0],
            send_sem=send_sems.at[0], recv_sem=recv_sems.at[0],
            device_id=(right,), device_id_type=pl.DeviceIdType.MESH,
        )
        rdma0.start()
        rdma0.wait()
        rbuf[0, :, :] = rbuf[0, :, :] + pchunk(lax.rem(me + N_DEV - 2, N_DEV))

        rdma1 = pltpu.make_async_remote_copy(
            src_ref=rbuf.at[0], dst_ref=rbuf.at[1],
            send_sem=send_sems.at[1], recv_sem=recv_sems.at[1],
            device_id=(right,), device_id_type=pl.DeviceIdType.MESH,
        )
        rdma1.start()
        rdma1.wait()
        rbuf[1, :, :] = rbuf[1, :, :] + pchunk(lax.rem(me + 1, N_DEV))

        rdma2 = pltpu.make_async_remote_copy(
            src_ref=rbuf.at[1], dst_ref=rbuf.at[2],
            send_sem=send_sems.at[2], recv_sem=recv_sems.at[2],
            device_id=(right,), device_id_type=pl.DeviceIdType.MESH,
        )
        rdma2.start()
        rdma2.wait()

        acc = rbuf[2, :, :] + pchunk(me)
        s = sx_ref[0] * sw_ref[0]
        out_ref[...] = jnp.maximum(acc * s, 0.0)

    return pl.pallas_call(
        body,
        out_shape=jax.ShapeDtypeStruct((m_per, n), jnp.float32),
        in_specs=[
            pl.BlockSpec(memory_space=pltpu.VMEM),
            pl.BlockSpec(memory_space=pltpu.VMEM),
            pl.BlockSpec(memory_space=pltpu.SMEM),
            pl.BlockSpec(memory_space=pltpu.SMEM),
        ],
        out_specs=pl.BlockSpec(memory_space=pltpu.VMEM),
        scratch_shapes=[
            pltpu.VMEM((3, m_per, n), jnp.float32),
            pltpu.VMEM((m_per, n), jnp.float32),
            pltpu.SemaphoreType.DMA((3,)),
            pltpu.SemaphoreType.DMA((3,)),
        ],
        compiler_params=pltpu.CompilerParams(collective_id=0),
    )(x, w_mat, scale_x, scale_w)


# baseline (device time: 180452 ns/iter reference)
import jax
import jax.numpy as jnp
from jax import lax
from jax.experimental import pallas as pl
from jax.experimental.pallas import tpu as pltpu

N_DEV = 4
M_CHUNK = 1024


def kernel(x, w_mat, scale_x, scale_w):
    m_tot, k_per = x.shape
    _, n = w_mat.shape
    m_per = m_tot // N_DEV

    def body(x_ref, w_ref, sx_ref, sw_ref, out_ref,
             rbuf, sbuf, send_sems, recv_sems):
        me = lax.axis_index("i")
        right = lax.rem(me + 1, N_DEV)
        left = lax.rem(me + N_DEV - 1, N_DEV)

        barrier_sem = pltpu.get_barrier_semaphore()
        for nbr in (left, right):
            pl.semaphore_signal(
                barrier_sem, inc=1,
                device_id=(nbr,), device_id_type=pl.DeviceIdType.MESH,
            )
        pl.semaphore_wait(barrier_sem, 2)

        w_bf = w_ref[...].astype(jnp.bfloat16)

        def pchunk(c):
            xa = x_ref[pl.ds(c * m_per, m_per), :].astype(jnp.bfloat16)
            return jnp.dot(xa, w_bf, preferred_element_type=jnp.float32)

        sbuf[...] = pchunk(lax.rem(me + N_DEV - 1, N_DEV)).astype(jnp.bfloat16)
        rdma0 = pltpu.make_async_remote_copy(
            src_ref=sbuf, dst_ref=rbuf.at[0],
            send_sem=send_sems.at[0], recv_sem=recv_sems.at[0],
            device_id=(right,), device_id_type=pl.DeviceIdType.MESH,
        )
        rdma0.start()
        rdma0.wait()
        rbuf[0, :, :] = (
            rbuf[0, :, :].astype(jnp.float32)
            + pchunk(lax.rem(me + N_DEV - 2, N_DEV))
        ).astype(jnp.bfloat16)

        rdma1 = pltpu.make_async_remote_copy(
            src_ref=rbuf.at[0], dst_ref=rbuf.at[1],
            send_sem=send_sems.at[1], recv_sem=recv_sems.at[1],
            device_id=(right,), device_id_type=pl.DeviceIdType.MESH,
        )
        rdma1.start()
        rdma1.wait()
        rbuf[1, :, :] = (
            rbuf[1, :, :].astype(jnp.float32) + pchunk(lax.rem(me + 1, N_DEV))
        ).astype(jnp.bfloat16)

        rdma2 = pltpu.make_async_remote_copy(
            src_ref=rbuf.at[1], dst_ref=rbuf.at[2],
            send_sem=send_sems.at[2], recv_sem=recv_sems.at[2],
            device_id=(right,), device_id_type=pl.DeviceIdType.MESH,
        )
        rdma2.start()
        rdma2.wait()

        acc = rbuf[2, :, :].astype(jnp.float32) + pchunk(me)
        s = sx_ref[0] * sw_ref[0]
        out_ref[...] = jnp.maximum(acc * s, 0.0)

    return pl.pallas_call(
        body,
        out_shape=jax.ShapeDtypeStruct((m_per, n), jnp.float32),
        in_specs=[
            pl.BlockSpec(memory_space=pltpu.VMEM),
            pl.BlockSpec(memory_space=pltpu.VMEM),
            pl.BlockSpec(memory_space=pltpu.SMEM),
            pl.BlockSpec(memory_space=pltpu.SMEM),
        ],
        out_specs=pl.BlockSpec(memory_space=pltpu.VMEM),
        scratch_shapes=[
            pltpu.VMEM((3, m_per, n), jnp.bfloat16),
            pltpu.VMEM((m_per, n), jnp.bfloat16),
            pltpu.SemaphoreType.DMA((3,)),
            pltpu.SemaphoreType.DMA((3,)),
        ],
        compiler_params=pltpu.CompilerParams(
            collective_id=0,
            vmem_limit_bytes=100 * 1024 * 1024,
        ),
    )(x, w_mat, scale_x, scale_w)


# device time: 101166 ns/iter; 1.7837x vs baseline; 1.7837x over previous
import jax
import jax.numpy as jnp
from jax import lax
from jax.experimental import pallas as pl
from jax.experimental.pallas import tpu as pltpu

N_DEV = 4


def kernel(x, w_mat, scale_x, scale_w):
    m_tot, k_per = x.shape
    _, n = w_mat.shape
    m_per = m_tot // N_DEV
    n2 = n // 2

    def body(x_ref, w_ref, sx_ref, sw_ref, out_ref,
             rbufA, rbufB, sbufA, sbufB, ssA, rsA, ssB, rsB):
        me = lax.axis_index("i")
        right = lax.rem(me + 1, N_DEV)
        left = lax.rem(me + N_DEV - 1, N_DEV)

        barrier_sem = pltpu.get_barrier_semaphore()
        for nbr in (left, right):
            pl.semaphore_signal(
                barrier_sem, inc=1,
                device_id=(nbr,), device_id_type=pl.DeviceIdType.MESH,
            )
        pl.semaphore_wait(barrier_sem, 2)

        def rows(c):
            r = lax.rem(c, N_DEV)
            return x_ref[pl.ds(r * m_per, m_per), :].astype(jnp.bfloat16)

        def gemm(xa, half):
            wb = w_ref[:, half * n2:(half + 1) * n2].astype(jnp.bfloat16)
            return jnp.dot(xa, wb, preferred_element_type=jnp.float32)

        def hopA(h, src):
            return pltpu.make_async_remote_copy(
                src_ref=src, dst_ref=rbufA.at[h],
                send_sem=ssA.at[h], recv_sem=rsA.at[h],
                device_id=(right,), device_id_type=pl.DeviceIdType.MESH,
            )

        def hopB(h, src):
            return pltpu.make_async_remote_copy(
                src_ref=src, dst_ref=rbufB.at[h],
                send_sem=ssB.at[h], recv_sem=rsB.at[h],
                device_id=(left,), device_id_type=pl.DeviceIdType.MESH,
            )

        sbufA[...] = gemm(rows(me + 3), 0).astype(jnp.bfloat16)
        sbufB[...] = gemm(rows(me + 1), 1).astype(jnp.bfloat16)
        rdmaA0 = hopA(0, sbufA)
        rdmaB0 = hopB(0, sbufB)
        rdmaA0.start()
        rdmaB0.start()

        x2 = rows(me + 2)
        g0A = gemm(x2, 0)
        g0B = gemm(x2, 1)

        rdmaA0.wait_recv()
        rbufA[0, :, :] = (rbufA[0, :, :].astype(jnp.float32) + g0A
                          ).astype(jnp.bfloat16)
        rdmaA1 = hopA(1, rbufA.at[0])
        rdmaA1.start()
        rdmaB0.wait_recv()
        rbufB[0, :, :] = (rbufB[0, :, :].astype(jnp.float32) + g0B
                          ).astype(jnp.bfloat16)
        rdmaB1 = hopB(1, rbufB.at[0])
        rdmaB1.start()

        g1A = gemm(rows(me + 1), 0)
        g1B = gemm(rows(me + 3), 1)

        rdmaA1.wait_recv()
        rbufA[1, :, :] = (rbufA[1, :, :].astype(jnp.float32) + g1A
                          ).astype(jnp.bfloat16)
        rdmaA2 = hopA(2, rbufA.at[1])
        rdmaA2.start()
        rdmaB1.wait_recv()
        rbufB[1, :, :] = (rbufB[1, :, :].astype(jnp.float32) + g1B
                          ).astype(jnp.bfloat16)
        rdmaB2 = hopB(2, rbufB.at[1])
        rdmaB2.start()

        xme = rows(me)
        g2A = gemm(xme, 0)
        g2B = gemm(xme, 1)
        s = sx_ref[0] * sw_ref[0]

        rdmaA2.wait_recv()
        out_ref[:, :n2] = jnp.maximum(
            (rbufA[2, :, :].astype(jnp.float32) + g2A) * s, 0.0)
        rdmaB2.wait_recv()
        out_ref[:, n2:] = jnp.maximum(
            (rbufB[2, :, :].astype(jnp.float32) + g2B) * s, 0.0)

        for r in (rdmaA0, rdmaB0, rdmaA1, rdmaB1, rdmaA2, rdmaB2):
            r.wait_send()

    return pl.pallas_call(
        body,
        out_shape=jax.ShapeDtypeStruct((m_per, n), jnp.float32),
        in_specs=[
            pl.BlockSpec(memory_space=pltpu.VMEM),
            pl.BlockSpec(memory_space=pltpu.VMEM),
            pl.BlockSpec(memory_space=pltpu.SMEM),
            pl.BlockSpec(memory_space=pltpu.SMEM),
        ],
        out_specs=pl.BlockSpec(memory_space=pltpu.VMEM),
        scratch_shapes=[
            pltpu.VMEM((3, m_per, n2), jnp.bfloat16),
            pltpu.VMEM((3, m_per, n2), jnp.bfloat16),
            pltpu.VMEM((m_per, n2), jnp.bfloat16),
            pltpu.VMEM((m_per, n2), jnp.bfloat16),
            pltpu.SemaphoreType.DMA((3,)),
            pltpu.SemaphoreType.DMA((3,)),
            pltpu.SemaphoreType.DMA((3,)),
            pltpu.SemaphoreType.DMA((3,)),
        ],
        compiler_params=pltpu.CompilerParams(
            collective_id=0,
            vmem_limit_bytes=100 * 1024 * 1024,
        ),
    )(x, w_mat, scale_x, scale_w)


# device time: 93903 ns/iter; 1.9217x vs baseline; 1.0773x over previous
import jax
import jax.numpy as jnp
from jax import lax
from jax.experimental import pallas as pl
from jax.experimental.pallas import tpu as pltpu

N_DEV = 4
N_RINGS = 4
RING_ORDER = (0, 2, 1, 3)


def kernel(x, w_mat, scale_x, scale_w):
    m_tot, k_per = x.shape
    _, n = w_mat.shape
    m_per = m_tot // N_DEV
    nq = n // N_RINGS

    def body(x_ref, w_ref, sx_ref, sw_ref, out_ref,
             rbuf, sbuf, ssems, rsems):
        me = lax.axis_index("i")
        right = lax.rem(me + 1, N_DEV)
        left = lax.rem(me + N_DEV - 1, N_DEV)

        barrier_sem = pltpu.get_barrier_semaphore()
        for nbr in (left, right):
            pl.semaphore_signal(
                barrier_sem, inc=1,
                device_id=(nbr,), device_id_type=pl.DeviceIdType.MESH,
            )
        pl.semaphore_wait(barrier_sem, 2)

        def rows(c):
            r = lax.rem(c + 2 * N_DEV, N_DEV)
            return x_ref[pl.ds(r * m_per, m_per), :].astype(jnp.bfloat16)

        def gemm(xa, r):
            wb = w_ref[:, r * nq:(r + 1) * nq].astype(jnp.bfloat16)
            return jnp.dot(xa, wb, preferred_element_type=jnp.float32)

        def rdma(r, h, src):
            tgt = right if r < 2 else left
            return pltpu.make_async_remote_copy(
                src_ref=src, dst_ref=rbuf.at[r, h],
                send_sem=ssems.at[r, h], recv_sem=rsems.at[r, h],
                device_id=(tgt,), device_id_type=pl.DeviceIdType.MESH,
            )

        started = {}

        xa_r = rows(me + 3)
        xa_l = None
        for r in RING_ORDER:
            if xa_l is None and r >= 2:
                xa_l = rows(me + 1)
            xa = xa_r if r < 2 else xa_l
            sbuf[r, :, :] = gemm(xa, r).astype(jnp.bfloat16)
            started[(r, 0)] = rdma(r, 0, sbuf.at[r])
            started[(r, 0)].start()

        for h in (0, 1):
            if h == 0:
                xa_r = xa_l = rows(me + 2)
            else:
                xa_r = rows(me + 1)
                xa_l = rows(me + 3)
            for r in RING_ORDER:
                g = gemm(xa_r if r < 2 else xa_l, r)
                started[(r, h)].wait_recv()
                rbuf[r, h, :, :] = (
                    rbuf[r, h, :, :].astype(jnp.float32) + g
                ).astype(jnp.bfloat16)
                started[(r, h + 1)] = rdma(r, h + 1, rbuf.at[r, h])
                started[(r, h + 1)].start()

        xa = rows(me)
        s = sx_ref[0] * sw_ref[0]
        for r in RING_ORDER:
            g = gemm(xa, r)
            started[(r, 2)].wait_recv()
            out_ref[:, r * nq:(r + 1) * nq] = jnp.maximum(
                (rbuf[r, 2, :, :].astype(jnp.float32) + g) * s, 0.0)

        for d in started.values():
            d.wait_send()

    return pl.pallas_call(
        body,
        out_shape=jax.ShapeDtypeStruct((m_per, n), jnp.float32),
        in_specs=[
            pl.BlockSpec(memory_space=pltpu.VMEM),
            pl.BlockSpec(memory_space=pltpu.VMEM),
            pl.BlockSpec(memory_space=pltpu.SMEM),
            pl.BlockSpec(memory_space=pltpu.SMEM),
        ],
        out_specs=pl.BlockSpec(memory_space=pltpu.VMEM),
        scratch_shapes=[
            pltpu.VMEM((N_RINGS, 3, m_per, nq), jnp.bfloat16),
            pltpu.VMEM((N_RINGS, m_per, nq), jnp.bfloat16),
            pltpu.SemaphoreType.DMA((N_RINGS, 3)),
            pltpu.SemaphoreType.DMA((N_RINGS, 3)),
        ],
        compiler_params=pltpu.CompilerParams(
            collective_id=0,
            vmem_limit_bytes=100 * 1024 * 1024,
        ),
    )(x, w_mat, scale_x, scale_w)
